# baseline (speedup 1.0000x reference)
import functools

import jax
import jax.numpy as jnp
from jax import lax
from jax.experimental import pallas as pl
from jax.experimental.pallas import tpu as pltpu
from jax.experimental.pallas import tpu_sc as plsc

NUM_ROWS = 1_000_000
DIM = 32
NUM_CORES = 2
NUM_SUBCORES = 16
NUM_WORKERS = NUM_CORES * NUM_SUBCORES

ALIGNED_COLS = (NUM_ROWS // 128) * 128
CHUNK_COLS = 1152
NFULL = ALIGNED_COLS // CHUNK_COLS
BASE_CHUNKS = NFULL // NUM_WORKERS
EXTRA_CHUNKS = NFULL - BASE_CHUNKS * NUM_WORKERS
TAIL_WORKER = EXTRA_CHUNKS
NBUF = 3

_MESH = plsc.VectorSubcoreMesh(core_axis_name="c", subcore_axis_name="s")


@functools.partial(
    pl.kernel,
    mesh=_MESH,
    out_type=jax.ShapeDtypeStruct((DIM, NUM_ROWS), jnp.float32),
    scratch_types=[
        pltpu.VMEM((DIM, CHUNK_COLS), jnp.float32),
        pltpu.VMEM((DIM, CHUNK_COLS), jnp.float32),
        pltpu.VMEM((DIM, CHUNK_COLS), jnp.float32),
        pltpu.VMEM((DIM, 128), jnp.float32),
        pltpu.SemaphoreType.DMA((NBUF,)),
        pltpu.SemaphoreType.DMA((NBUF,)),
        pltpu.SemaphoreType.DMA,
    ],
)
def _copy_kernel(in_hbm, out_hbm, buf0, buf1, buf2, tail_buf, in_sems, out_sems, tail_sem):
    wid = lax.axis_index("s") * NUM_CORES + lax.axis_index("c")
    bufs = (buf0, buf1, buf2)

    def col_start(k):
        return pl.multiple_of((wid + k * NUM_WORKERS) * CHUNK_COLS, 128)

    def copy_in(k):
        return pltpu.make_async_copy(
            in_hbm.at[:, pl.ds(col_start(k), CHUNK_COLS)],
            bufs[k % NBUF],
            in_sems.at[k % NBUF],
        )

    def copy_out(k):
        return pltpu.make_async_copy(
            bufs[k % NBUF],
            out_hbm.at[:, pl.ds(col_start(k), CHUNK_COLS)],
            out_sems.at[k % NBUF],
        )

    for j in range(NBUF):
        copy_in(j).start()
    for k in range(BASE_CHUNKS):
        copy_in(k).wait()
        copy_out(k).start()
        if k + NBUF < BASE_CHUNKS:
            copy_out(k).wait()
            copy_in(k + NBUF).start()
    for k in range(max(0, BASE_CHUNKS - NBUF), BASE_CHUNKS):
        copy_out(k).wait()

    @pl.when(wid < EXTRA_CHUNKS)
    def _extra():
        start = pl.multiple_of((BASE_CHUNKS * NUM_WORKERS + wid) * CHUNK_COLS, 128)
        pltpu.make_async_copy(
            in_hbm.at[:, pl.ds(start, CHUNK_COLS)], buf0, tail_sem
        ).start()
        pltpu.make_async_copy(
            in_hbm.at[:, pl.ds(start, CHUNK_COLS)], buf0, tail_sem
        ).wait()
        pltpu.make_async_copy(
            buf0, out_hbm.at[:, pl.ds(start, CHUNK_COLS)], tail_sem
        ).start()
        pltpu.make_async_copy(
            buf0, out_hbm.at[:, pl.ds(start, CHUNK_COLS)], tail_sem
        ).wait()

    @pl.when(wid == TAIL_WORKER)
    def _tail():
        last = pl.multiple_of(ALIGNED_COLS + wid * 0, 128)
        pltpu.make_async_copy(
            in_hbm.at[:, pl.ds(last, 128)], tail_buf, tail_sem
        ).start()
        pltpu.make_async_copy(
            in_hbm.at[:, pl.ds(last, 128)], tail_buf, tail_sem
        ).wait()
        pltpu.make_async_copy(
            tail_buf, out_hbm.at[:, pl.ds(last, 128)], tail_sem
        ).start()
        pltpu.make_async_copy(
            tail_buf, out_hbm.at[:, pl.ds(last, 128)], tail_sem
        ).wait()


def kernel(embedding):
    return _copy_kernel(embedding.T).T


# speedup vs baseline: 1.0162x; 1.0162x over previous
import functools

import jax
import jax.numpy as jnp
from jax import lax
from jax.experimental import pallas as pl
from jax.experimental.pallas import tpu as pltpu
from jax.experimental.pallas import tpu_sc as plsc

NUM_ROWS = 1_000_000
DIM = 32
NUM_CORES = 2
NUM_SUBCORES = 16
NUM_WORKERS = NUM_CORES * NUM_SUBCORES

ALIGNED_COLS = (NUM_ROWS // 128) * 128
CHUNK_COLS = 1792
NFULL = ALIGNED_COLS // CHUNK_COLS
BASE_CHUNKS = NFULL // NUM_WORKERS
EXTRA_CHUNKS = NFULL - BASE_CHUNKS * NUM_WORKERS
TAIL_WORKER = EXTRA_CHUNKS
PIPE_CHUNKS = BASE_CHUNKS + 1
NBUF = 2

_MESH = plsc.VectorSubcoreMesh(core_axis_name="c", subcore_axis_name="s")


@functools.partial(
    pl.kernel,
    mesh=_MESH,
    out_type=jax.ShapeDtypeStruct((DIM, NUM_ROWS), jnp.float32),
    scratch_types=[
        pltpu.VMEM((DIM, CHUNK_COLS), jnp.float32),
        pltpu.VMEM((DIM, CHUNK_COLS), jnp.float32),
        pltpu.VMEM((DIM, 128), jnp.float32),
        pltpu.SemaphoreType.DMA((NBUF,)),
        pltpu.SemaphoreType.DMA((NBUF,)),
        pltpu.SemaphoreType.DMA,
    ],
)
def _copy_kernel(in_hbm, out_hbm, buf0, buf1, tail_buf, in_sems, out_sems, tail_sem):
    wid = lax.axis_index("s") * NUM_CORES + lax.axis_index("c")
    bufs = (buf0, buf1)

    def col_start(k):
        return pl.multiple_of((wid + k * NUM_WORKERS) * CHUNK_COLS, 128)

    def copy_in(k):
        return pltpu.make_async_copy(
            in_hbm.at[:, pl.ds(col_start(k), CHUNK_COLS)],
            bufs[k % NBUF],
            in_sems.at[k % NBUF],
        )

    def copy_out(k):
        return pltpu.make_async_copy(
            bufs[k % NBUF],
            out_hbm.at[:, pl.ds(col_start(k), CHUNK_COLS)],
            out_sems.at[k % NBUF],
        )

    def guarded(k, op):
        if k < BASE_CHUNKS:
            op()
        else:
            pl.when(wid < EXTRA_CHUNKS)(op)

    def tail_in():
        last = pl.multiple_of(ALIGNED_COLS + wid * 0, 128)
        return pltpu.make_async_copy(
            in_hbm.at[:, pl.ds(last, 128)], tail_buf, tail_sem
        )

    def tail_out():
        last = pl.multiple_of(ALIGNED_COLS + wid * 0, 128)
        return pltpu.make_async_copy(
            tail_buf, out_hbm.at[:, pl.ds(last, 128)], tail_sem
        )

    pl.when(wid == TAIL_WORKER)(lambda: tail_in().start())

    for j in range(NBUF):
        guarded(j, lambda j=j: copy_in(j).start())
    for k in range(PIPE_CHUNKS):
        guarded(k, lambda k=k: copy_in(k).wait())
        guarded(k, lambda k=k: copy_out(k).start())
        if k + NBUF < PIPE_CHUNKS:
            guarded(k, lambda k=k: copy_out(k).wait())
            guarded(k + NBUF, lambda k=k: copy_in(k + NBUF).start())
    for k in range(max(0, PIPE_CHUNKS - NBUF), PIPE_CHUNKS):
        guarded(k, lambda k=k: copy_out(k).wait())

    @pl.when(wid == TAIL_WORKER)
    def _tail():
        tail_in().wait()
        tail_out().start()
        tail_out().wait()


def kernel(embedding):
    return _copy_kernel(embedding.T).T


# speedup vs baseline: 1.0373x; 1.0208x over previous
import functools

import jax
import jax.numpy as jnp
from jax import lax
from jax.experimental import pallas as pl
from jax.experimental.pallas import tpu as pltpu
from jax.experimental.pallas import tpu_sc as plsc

NUM_ROWS = 1_000_000
DIM = 32
NUM_CORES = 2
NUM_SUBCORES = 16
NUM_WORKERS = NUM_CORES * NUM_SUBCORES

LANE = 128
TILES = NUM_ROWS // LANE
TILES_PER_WORKER = TILES // NUM_WORKERS
EXTRA_TILES = TILES - TILES_PER_WORKER * NUM_WORKERS
TAIL_WORKER = EXTRA_TILES

CHUNK_TILES = [15] * 16 + [4]
CHUNK_OFFS = [sum(CHUNK_TILES[:k]) for k in range(len(CHUNK_TILES))]
NCHUNKS = len(CHUNK_TILES)
MAXC = max(CHUNK_TILES) * LANE
NBUF = 2

_MESH = plsc.VectorSubcoreMesh(core_axis_name="c", subcore_axis_name="s")


@functools.partial(
    pl.kernel,
    mesh=_MESH,
    out_type=jax.ShapeDtypeStruct((DIM, NUM_ROWS), jnp.float32),
    scratch_types=[
        pltpu.VMEM((DIM, MAXC), jnp.float32),
        pltpu.VMEM((DIM, MAXC), jnp.float32),
        pltpu.VMEM((DIM, LANE), jnp.float32),
        pltpu.SemaphoreType.DMA((NBUF,)),
        pltpu.SemaphoreType.DMA((NBUF,)),
        pltpu.SemaphoreType.DMA,
    ],
)
def _copy_kernel(in_hbm, out_hbm, buf0, buf1, tail_buf, in_sems, out_sems, tail_sem):
    wid = lax.axis_index("s") * NUM_CORES + lax.axis_index("c")
    base = wid * (TILES_PER_WORKER * LANE)
    bufs = (buf0, buf1)

    def hbm_slice(ref, k):
        start = pl.multiple_of(base + CHUNK_OFFS[k] * LANE, LANE)
        return ref.at[:, pl.ds(start, CHUNK_TILES[k] * LANE)]

    def vmem_buf(k):
        b = bufs[k % NBUF]
        cols = CHUNK_TILES[k] * LANE
        return b if cols == MAXC else b.at[:, :cols]

    def copy_in(k):
        return pltpu.make_async_copy(
            hbm_slice(in_hbm, k), vmem_buf(k), in_sems.at[k % NBUF]
        )

    def copy_out(k):
        return pltpu.make_async_copy(
            vmem_buf(k), hbm_slice(out_hbm, k), out_sems.at[k % NBUF]
        )

    def tail_slice(ref):
        extra = (TILES_PER_WORKER * NUM_WORKERS + wid) * LANE
        start = pl.multiple_of(jnp.where(wid == TAIL_WORKER, TILES * LANE, extra), LANE)
        return ref.at[:, pl.ds(start, LANE)]

    has_tail = wid <= TAIL_WORKER

    pl.when(has_tail)(
        lambda: pltpu.make_async_copy(tail_slice(in_hbm), tail_buf, tail_sem).start()
    )

    for j in range(NBUF):
        copy_in(j).start()
    for k in range(NCHUNKS):
        copy_in(k).wait()
        copy_out(k).start()
        if k + NBUF < NCHUNKS:
            copy_out(k).wait()
            copy_in(k + NBUF).start()
    for k in range(max(0, NCHUNKS - NBUF), NCHUNKS):
        copy_out(k).wait()

    @pl.when(has_tail)
    def _tail():
        pltpu.make_async_copy(tail_slice(in_hbm), tail_buf, tail_sem).wait()
        pltpu.make_async_copy(tail_buf, tail_slice(out_hbm), tail_sem).start()
        pltpu.make_async_copy(tail_buf, tail_slice(out_hbm), tail_sem).wait()


def kernel(embedding):
    return _copy_kernel(embedding.T).T


# speedup vs baseline: 1.0932x; 1.0538x over previous
import functools

import jax
import jax.numpy as jnp
from jax import lax
from jax.experimental import pallas as pl
from jax.experimental.pallas import tpu as pltpu
from jax.experimental.pallas import tpu_sc as plsc

NUM_ROWS = 1_000_000
DIM = 32
NUM_CORES = 2
NUM_SUBCORES = 16
NUM_WORKERS = NUM_CORES * NUM_SUBCORES

LANE = 128
TILES = NUM_ROWS // LANE
TILES_PER_WORKER = TILES // NUM_WORKERS
EXTRA_TILES = TILES - TILES_PER_WORKER * NUM_WORKERS
TAIL_WORKER = EXTRA_TILES

CHUNK_TILES = [15] * 16 + [4]
CHUNK_OFFS = [sum(CHUNK_TILES[:k]) for k in range(len(CHUNK_TILES))]
NCHUNKS = len(CHUNK_TILES)
MAXC = max(CHUNK_TILES) * LANE
NBUF = 2

_MESH = plsc.VectorSubcoreMesh(core_axis_name="c", subcore_axis_name="s")


@functools.partial(
    pl.kernel,
    mesh=_MESH,
    out_type=jax.ShapeDtypeStruct((DIM, NUM_ROWS), jnp.float32),
    scratch_types=[
        pltpu.VMEM_SHARED((NUM_SUBCORES, DIM, MAXC), jnp.float32),
        pltpu.VMEM_SHARED((NUM_SUBCORES, DIM, MAXC), jnp.float32),
        pltpu.VMEM((DIM, LANE), jnp.float32),
        pltpu.SemaphoreType.DMA((NBUF,)),
        pltpu.SemaphoreType.DMA((NBUF,)),
        pltpu.SemaphoreType.DMA,
    ],
)
def _copy_kernel(in_hbm, out_hbm, buf0, buf1, tail_buf, in_sems, out_sems, tail_sem):
    wid = lax.axis_index("s") * NUM_CORES + lax.axis_index("c")
    sid = lax.axis_index("s")
    base = wid * (TILES_PER_WORKER * LANE)
    bufs = (buf0.at[sid], buf1.at[sid])

    def hbm_slice(ref, k):
        start = pl.multiple_of(base + CHUNK_OFFS[k] * LANE, LANE)
        return ref.at[:, pl.ds(start, CHUNK_TILES[k] * LANE)]

    def vmem_buf(k):
        b = bufs[k % NBUF]
        cols = CHUNK_TILES[k] * LANE
        return b if cols == MAXC else b.at[:, :cols]

    def copy_in(k):
        return pltpu.make_async_copy(
            hbm_slice(in_hbm, k), vmem_buf(k), in_sems.at[k % NBUF]
        )

    def copy_out(k):
        return pltpu.make_async_copy(
            vmem_buf(k), hbm_slice(out_hbm, k), out_sems.at[k % NBUF]
        )

    def tail_slice(ref):
        extra = (TILES_PER_WORKER * NUM_WORKERS + wid) * LANE
        start = pl.multiple_of(jnp.where(wid == TAIL_WORKER, TILES * LANE, extra), LANE)
        return ref.at[:, pl.ds(start, LANE)]

    has_tail = wid <= TAIL_WORKER

    pl.when(has_tail)(
        lambda: pltpu.make_async_copy(tail_slice(in_hbm), tail_buf, tail_sem).start()
    )

    for j in range(NBUF):
        copy_in(j).start()
    for k in range(NCHUNKS):
        copy_in(k).wait()
        copy_out(k).start()
        if k + NBUF < NCHUNKS:
            copy_out(k).wait()
            copy_in(k + NBUF).start()
    for k in range(max(0, NCHUNKS - NBUF), NCHUNKS):
        copy_out(k).wait()

    @pl.when(has_tail)
    def _tail():
        pltpu.make_async_copy(tail_slice(in_hbm), tail_buf, tail_sem).wait()
        pltpu.make_async_copy(tail_buf, tail_slice(out_hbm), tail_sem).start()
        pltpu.make_async_copy(tail_buf, tail_slice(out_hbm), tail_sem).wait()


def kernel(embedding):
    return _copy_kernel(embedding.T).T
